# baseline (speedup 1.0000x reference)
import functools

import jax
import jax.numpy as jnp
from jax import lax
from jax.experimental import pallas as pl
from jax.experimental.pallas import tpu as pltpu
from jax.experimental.pallas import tpu_sc as plsc

B = 16384
D = 16
DB = 128
N_OUT = 1024
NC, NS = 2, 16
NW = NC * NS
B_PER_W = B // NW
CHUNK = 128
NCH = B_PER_W // CHUNK

_sc_mesh = plsc.VectorSubcoreMesh(core_axis_name="c", subcore_axis_name="s")


@functools.partial(
    pl.kernel,
    mesh=_sc_mesh,
    out_type=jax.ShapeDtypeStruct((NW, NCH, CHUNK, DB), jnp.float32),
    scratch_types=[
        pltpu.VMEM((NCH, CHUNK), jnp.int32),
        pltpu.VMEM((NCH, CHUNK, DB), jnp.float32),
        pltpu.SemaphoreType.DMA,
    ],
)
def _sc_gather(idx_hbm, table_hbm, out_hbm, idx_v, rows_v, sem):
    wid = lax.axis_index("s") * NC + lax.axis_index("c")
    pltpu.sync_copy(idx_hbm.at[wid], idx_v)
    copies = []
    for j in range(NCH):
        copies.append(
            pltpu.async_copy(table_hbm.at[idx_v.at[j]], rows_v.at[j], sem)
        )
    for cp in copies:
        cp.wait()
    pltpu.sync_copy(rows_v, out_hbm.at[wid])


def _mm_body(w_ref, x_ref, sub_ref, b_ref, o_ref):
    sub = sub_ref[...]
    col_j = lax.broadcasted_iota(jnp.int32, (1, DB), 1) // D
    x = jnp.where(col_j == sub, x_ref[...], 0.0)
    o_ref[...] = (
        lax.dot_general(
            w_ref[...], x, (((0,), (1,)), ((), ())),
            preferred_element_type=jnp.float32,
        )
        + b_ref[...]
    )


def _tc_matmul(w, x, sub, b_col, block_m=1024):
    m = x.shape[0]
    return pl.pallas_call(
        _mm_body,
        grid=(m // block_m,),
        in_specs=[
            pl.BlockSpec((DB, N_OUT), lambda i: (0, 0)),
            pl.BlockSpec((block_m, DB), lambda i: (i, 0)),
            pl.BlockSpec((block_m, 1), lambda i: (i, 0)),
            pl.BlockSpec((N_OUT, 1), lambda i: (0, 0)),
        ],
        out_specs=pl.BlockSpec((N_OUT, block_m), lambda i: (0, i)),
        out_shape=jax.ShapeDtypeStruct((N_OUT, m), jnp.float32),
    )(w, x, sub, b_col)


def kernel(inputs, emb_table, dense_w, dense_b):
    idx = inputs.reshape(B).astype(jnp.int32)
    blk_idx = (idx // 8).reshape(NW, NCH, CHUNK)
    sub = (idx % 8).reshape(B, 1)
    table128 = emb_table.reshape(125000, DB)
    blocks = _sc_gather(blk_idx, table128)
    w128 = jnp.tile(dense_w, (8, 1))
    out_t = _tc_matmul(
        w128, blocks.reshape(B, DB), sub, dense_b.reshape(N_OUT, 1)
    )
    return out_t.T.reshape(B, 4, 4, 64)
